# Initial kernel scaffold; baseline (speedup 1.0000x reference)
import jax
import jax.numpy as jnp
from jax.experimental import pallas as pl


def kernel(conv0_w, conv0_b, conv1_w, conv1_b, conv2_w, conv2_b, conv3_w, conv3_b, conv4_w, conv4_b, fc0_w, fc0_b, fc1_w, fc1_b, fc2_w, fc2_b, x):
    raise NotImplementedError("write your pallas kernel here")


# speedup vs baseline: 32.9510x; 32.9510x over previous
import functools

import jax
import jax.numpy as jnp
from jax.experimental import pallas as pl
from jax.experimental.pallas import tpu as pltpu


def _pool3s2(y):
    B, H, W, C = y.shape
    po = (H - 3) // 2 + 1
    y = jnp.pad(y, ((0, 0), (0, H % 2), (0, 0), (0, 0)))
    y = y.reshape(B, (H + 1) // 2, 2, W, C)
    e, o = y[:, :, 0], y[:, :, 1]
    y = jnp.maximum(jnp.maximum(e[:, :po], o[:, :po]), e[:, 1:po + 1])
    y = jnp.pad(y, ((0, 0), (0, 0), (0, W % 2), (0, 0)))
    y = y.reshape(B, po, (W + 1) // 2, 2, C)
    e, o = y[:, :, :, 0], y[:, :, :, 1]
    return jnp.maximum(jnp.maximum(e[:, :, :po], o[:, :, :po]),
                       e[:, :, 1:po + 1])


def _fused_conv_kernel(x_ref, w_ref, b_ref, o_ref, *, k, ho, wo, pool,
                       flatten):
    x = x_ref[...]
    B = x.shape[0]
    C = x.shape[3]
    pieces = []
    for i in range(k):
        for j in range(k):
            pieces.append(x[:, i:i + ho, j:j + wo, :].reshape(B * ho * wo, C))
    patches = jnp.concatenate(pieces, axis=1)
    y = jnp.dot(patches, w_ref[...], preferred_element_type=jnp.float32)
    y = jnp.maximum(y + b_ref[...], 0.0)
    cout = y.shape[1]
    y = y.reshape(B, ho, wo, cout)
    if pool:
        y = _pool3s2(y)
        if flatten:
            po = y.shape[1]
            y = y.reshape(B, po * po, cout)
            y = jnp.transpose(y, (0, 2, 1)).reshape(B, po * po * cout)
    o_ref[...] = y.astype(o_ref.dtype)


def _fused_conv(x, w, b, *, k, blk, pool=False, flatten=False):
    N, Hp, Wp, C = x.shape
    ho, wo = Hp - k + 1, Wp - k + 1
    cout = w.shape[1]
    if pool:
        so = (ho - 3) // 2 + 1
        out_shape = ((N, so * so * cout) if flatten
                     else (N, so, so, cout))
    else:
        out_shape = (N, ho, wo, cout)
    ospec = (pl.BlockSpec((blk, out_shape[1]), lambda i: (i, 0))
             if len(out_shape) == 2 else
             pl.BlockSpec((blk,) + out_shape[1:], lambda i: (i, 0, 0, 0)))
    return pl.pallas_call(
        functools.partial(_fused_conv_kernel, k=k, ho=ho, wo=wo, pool=pool,
                          flatten=flatten),
        out_shape=jax.ShapeDtypeStruct(out_shape, jnp.bfloat16),
        grid=(N // blk,),
        in_specs=[
            pl.BlockSpec((blk, Hp, Wp, C), lambda i: (i, 0, 0, 0)),
            pl.BlockSpec(w.shape, lambda i: (0, 0)),
            pl.BlockSpec(b.shape, lambda i: (0, 0)),
        ],
        out_specs=ospec,
        compiler_params=pltpu.CompilerParams(
            dimension_semantics=("parallel",)),
    )(x, w, b)


def _fc_kernel(x_ref, w_ref, b_ref, o_ref, *, relu):
    y = jnp.dot(x_ref[...], w_ref[...], preferred_element_type=jnp.float32)
    y = y + b_ref[...]
    if relu:
        y = jnp.maximum(y, 0.0)
    o_ref[...] = y.astype(o_ref.dtype)


def _fc(x, w, b, *, tn, relu, out_dtype):
    M, K = x.shape
    Np = w.shape[1]
    return pl.pallas_call(
        functools.partial(_fc_kernel, relu=relu),
        out_shape=jax.ShapeDtypeStruct((M, Np), out_dtype),
        grid=(Np // tn,),
        in_specs=[
            pl.BlockSpec((M, K), lambda j: (0, 0)),
            pl.BlockSpec((K, tn), lambda j: (0, j)),
            pl.BlockSpec((1, tn), lambda j: (0, j)),
        ],
        out_specs=pl.BlockSpec((M, tn), lambda j: (0, j)),
        compiler_params=pltpu.CompilerParams(
            dimension_semantics=("parallel",)),
    )(x, w, b)


def _pad_hw(a, p):
    return jnp.pad(a, ((0, 0), (p, p), (p, p), (0, 0)))


def kernel(conv0_w, conv0_b, conv1_w, conv1_b, conv2_w, conv2_b, conv3_w,
           conv3_b, conv4_w, conv4_b, fc0_w, fc0_b, fc1_w, fc1_b, fc2_w,
           fc2_b, x):
    N = x.shape[0]

    xt = jnp.transpose(x, (0, 2, 3, 1))
    xp = jnp.pad(xt, ((0, 0), (2, 2), (2, 2), (0, 0)))
    xs = (xp.reshape(N, 57, 4, 57, 4, 3).transpose(0, 1, 3, 2, 4, 5)
          .reshape(N, 57, 57, 48).astype(jnp.bfloat16))
    w0 = conv0_w[:363, :64].reshape(11, 11, 3, 64)
    w0 = jnp.pad(w0, ((0, 1), (0, 1), (0, 0), (0, 0)))
    w0 = (w0.reshape(3, 4, 3, 4, 3, 64).transpose(0, 2, 1, 3, 4, 5)
          .reshape(432, 64))

    a = _fused_conv(xs, w0, conv0_b[:, :64], k=3, blk=1, pool=True)
    a = _fused_conv(_pad_hw(a, 2), conv1_w[:1600, :192], conv1_b[:, :192],
                    k=5, blk=2, pool=True)
    a = _fused_conv(_pad_hw(a, 1), conv2_w[:1728, :384], conv2_b[:, :384],
                    k=3, blk=4)
    a = _fused_conv(_pad_hw(a, 1), conv3_w[:3456, :256], conv3_b[:, :256],
                    k=3, blk=4)
    f = _fused_conv(_pad_hw(a, 1), conv4_w[:2304, :256], conv4_b[:, :256],
                    k=3, blk=8, pool=True, flatten=True)

    h = _fc(f, fc0_w, fc0_b, tn=512, relu=True, out_dtype=jnp.bfloat16)
    h = _fc(h, fc1_w, fc1_b, tn=512, relu=True, out_dtype=jnp.bfloat16)
    out = _fc(h, fc2_w, fc2_b, tn=256, relu=False, out_dtype=jnp.float32)
    return out[:, :1000]
